# Initial kernel scaffold; baseline (speedup 1.0000x reference)
import jax
import jax.numpy as jnp
from jax.experimental import pallas as pl


def kernel(x, edge_index_0, edge_index_1, edge_attr_0, edge_attr_1, u, batch, We0, be0, We1, be1, Wn, bn, Wg, bg):
    raise NotImplementedError("write your pallas kernel here")


# speedup vs baseline: 5.5835x; 5.5835x over previous
import functools

import jax
import jax.numpy as jnp
from jax import lax
from jax.experimental import pallas as pl
from jax.experimental.pallas import tpu as pltpu
from jax.experimental.pallas import tpu_sc as plsc

_NC = 2
_NS = 16
_NW = _NC * _NS

_SUB = 80
_K = 5
_CHUNK = _SUB * _K


def _prep_body(x_ref, w_ref, c_ref, p0_ref, p1_ref, p2_ref, p3_ref):
    p = jnp.dot(x_ref[...], w_ref[...], preferred_element_type=jnp.float32)
    p = p + c_ref[...]
    p0_ref[...] = p[:, 0:16]
    p1_ref[...] = p[:, 16:32]
    p2_ref[...] = p[:, 32:48]
    p3_ref[...] = p[:, 48:64]


def _amat_body(ea0_ref, ea1_ref, w0_ref, w1_ref, a0_ref, a1_ref):
    a0_ref[...] = jnp.dot(ea0_ref[...], w0_ref[...],
                          preferred_element_type=jnp.float32)
    a1_ref[...] = jnp.dot(ea1_ref[...], w1_ref[...],
                          preferred_element_type=jnp.float32)


def _node_body(nblk, n_total, x_ref, a0_ref, a1_ref, wx_ref, wa_ref, cn_ref,
               wg_ref, u_ref, bg_ref, xn_ref, un_ref, acc_ref):
    i = pl.program_id(0)
    agg = jnp.concatenate(
        [a0_ref[0] + a0_ref[1], a1_ref[0] + a1_ref[1]], axis=-1)
    xb = jnp.dot(x_ref[...], wx_ref[...], preferred_element_type=jnp.float32)
    xb = xb + jnp.dot(agg, wa_ref[...], preferred_element_type=jnp.float32)
    xb = xb + cn_ref[...]
    xn_ref[...] = xb

    @pl.when(i == 0)
    def _():
        acc_ref[...] = jnp.zeros_like(acc_ref)

    acc_ref[...] += jnp.sum(xb, axis=0, keepdims=True)

    @pl.when(i == nblk - 1)
    def _():
        mean = acc_ref[...] / jnp.float32(n_total)
        un = jnp.dot(mean, wg_ref[0:128, :],
                     preferred_element_type=jnp.float32)
        un = un + jnp.dot(u_ref[...], wg_ref[128:160, :],
                          preferred_element_type=jnp.float32)
        un_ref[...] = un + bg_ref[...]


def _sc_body(n_nodes, g_chunks,
             pr0, pc0, pr1, pc1, a0h, a1h, row0h, col0h, row1h, col1h,
             e0h, e1h, ag0h, ag1h,
             rowb, colb, rb, cb, ab, eb, cpb, ag0s, ag1s, sem):
    cid = lax.axis_index("c")
    sid = lax.axis_index("s")
    wid = cid * _NS + sid

    small = (n_nodes // _NS) // 8 * 8
    big = n_nodes - small * (_NS - 1)
    tile_lo = sid * small
    last = sid == _NS - 1

    z = jnp.zeros((16,), jnp.float32)

    def zbody(i, c):
        cpb[i] = z
        return c

    lax.fori_loop(0, big, zbody, 0)

    @pl.when(last)
    def _():
        pltpu.sync_copy(cpb, ag0s.at[pl.ds(tile_lo, big)])
        pltpu.sync_copy(cpb, ag1s.at[pl.ds(tile_lo, big)])

    @pl.when(jnp.logical_not(last))
    def _():
        pltpu.sync_copy(cpb.at[pl.ds(0, small)], ag0s.at[pl.ds(tile_lo, small)])
        pltpu.sync_copy(cpb.at[pl.ds(0, small)], ag1s.at[pl.ds(tile_lo, small)])

    plsc.subcore_barrier()

    for prh, pch, ah, rowh, colh, eh, ags in (
        (pr0, pc0, a0h, row0h, col0h, e0h, ag0s),
        (pr1, pc1, a1h, row1h, col1h, e1h, ag1s),
    ):
        base_chunk = wid * g_chunks

        def chunk(g, c, prh=prh, pch=pch, ah=ah, rowh=rowh, colh=colh,
                  eh=eh, ags=ags, base_chunk=base_chunk):
            cidx = base_chunk + g
            rbase = cidx * _K
            pltpu.sync_copy(rowh.at[cidx], rowb)
            pltpu.sync_copy(colh.at[cidx], colb)
            pltpu.sync_copy(ah.at[pl.ds(rbase, _K)], ab)
            descs = []
            for s in range(_K):
                descs.append(
                    pltpu.async_copy(prh.at[rowb.at[s]], rb.at[s], sem))
                descs.append(
                    pltpu.async_copy(pch.at[colb.at[s]], cb.at[s], sem))
            for d in descs:
                d.wait()

            def ebody(i, cc):
                for s in range(_K):
                    v = rb[s, i] + cb[s, i] + ab[s, i]
                    eb[s, i] = jnp.maximum(v, 0.0)
                return cc

            lax.fori_loop(0, _SUB, ebody, 0)
            pltpu.sync_copy(eb, eh.at[pl.ds(rbase, _K)])
            for s in range(_K):
                pltpu.sync_copy(eb.at[s], ags.at[colb.at[s]], add=True)
            return c

        lax.fori_loop(0, g_chunks, chunk, 0)

    plsc.subcore_barrier()

    @pl.when(last)
    def _():
        pltpu.sync_copy(ag0s.at[pl.ds(tile_lo, big)], cpb)
        pltpu.sync_copy(cpb, ag0h.at[cid, pl.ds(tile_lo, big)])
        pltpu.sync_copy(ag1s.at[pl.ds(tile_lo, big)], cpb)
        pltpu.sync_copy(cpb, ag1h.at[cid, pl.ds(tile_lo, big)])

    @pl.when(jnp.logical_not(last))
    def _():
        cps = cpb.at[pl.ds(0, small)]
        pltpu.sync_copy(ag0s.at[pl.ds(tile_lo, small)], cps)
        pltpu.sync_copy(cps, ag0h.at[cid, pl.ds(tile_lo, small)])
        pltpu.sync_copy(ag1s.at[pl.ds(tile_lo, small)], cps)
        pltpu.sync_copy(cps, ag1h.at[cid, pl.ds(tile_lo, small)])


def kernel(x, edge_index_0, edge_index_1, edge_attr_0, edge_attr_1, u, batch,
           We0, be0, We1, be1, Wn, bn, Wg, bg):
    n, d = x.shape
    e = edge_index_0.shape[1]
    de = edge_attr_0.shape[1]
    du = u.shape[1]
    f32 = jnp.float32

    u0 = u[0]
    wcat = jnp.concatenate(
        [We0[:d], We0[d:2 * d], We1[:d], We1[d:2 * d]], axis=1)
    c0 = u0 @ We0[2 * d + de:] + be0
    c1 = u0 @ We1[2 * d + de:] + be1
    zc = jnp.zeros_like(c0)
    cc = jnp.concatenate([zc, c0, zc, c1])[None, :]
    w8_0 = jnp.kron(jnp.eye(8, dtype=f32), We0[2 * d:2 * d + de])
    w8_1 = jnp.kron(jnp.eye(8, dtype=f32), We1[2 * d:2 * d + de])
    wnx = Wn[:d]
    wna = Wn[d:d + 2 * de]
    cn = (u0 @ Wn[d + 2 * de:] + bn)[None, :]

    tbl_shape = jax.ShapeDtypeStruct((n, de), f32)
    pr0, pc0, pr1, pc1 = pl.pallas_call(
        _prep_body,
        out_shape=(tbl_shape, tbl_shape, tbl_shape, tbl_shape),
    )(x, wcat, cc)

    ea0r = edge_attr_0.reshape(e // 8, 8 * de)
    ea1r = edge_attr_1.reshape(e // 8, 8 * de)
    blk = 2000
    nblk_e = (e // 8) // blk
    a0r, a1r = pl.pallas_call(
        _amat_body,
        grid=(nblk_e,),
        in_specs=[
            pl.BlockSpec((blk, 8 * de), lambda i: (i, 0)),
            pl.BlockSpec((blk, 8 * de), lambda i: (i, 0)),
            pl.BlockSpec((8 * de, 8 * de), lambda i: (0, 0)),
            pl.BlockSpec((8 * de, 8 * de), lambda i: (0, 0)),
        ],
        out_specs=(
            pl.BlockSpec((blk, 8 * de), lambda i: (i, 0)),
            pl.BlockSpec((blk, 8 * de), lambda i: (i, 0)),
        ),
        out_shape=(
            jax.ShapeDtypeStruct((e // 8, 8 * de), f32),
            jax.ShapeDtypeStruct((e // 8, 8 * de), f32),
        ),
    )(ea0r, ea1r, w8_0, w8_1)

    nrows = e // _SUB
    g_chunks = nrows // _NW // _K
    nchunks = _NW * g_chunks
    row0 = edge_index_0[0].reshape(nchunks, _K, _SUB)
    col0 = edge_index_0[1].reshape(nchunks, _K, _SUB)
    row1 = edge_index_1[0].reshape(nchunks, _K, _SUB)
    col1 = edge_index_1[1].reshape(nchunks, _K, _SUB)
    a0h = a0r.reshape(nrows, _SUB, de)
    a1h = a1r.reshape(nrows, _SUB, de)

    mesh = plsc.VectorSubcoreMesh(
        core_axis_name="c", subcore_axis_name="s",
        num_cores=_NC, num_subcores=_NS)
    sc_fn = pl.kernel(
        functools.partial(_sc_body, n, g_chunks),
        out_type=(
            jax.ShapeDtypeStruct((nrows, _SUB, de), f32),
            jax.ShapeDtypeStruct((nrows, _SUB, de), f32),
            jax.ShapeDtypeStruct((_NC, n, de), f32),
            jax.ShapeDtypeStruct((_NC, n, de), f32),
        ),
        mesh=mesh,
        scratch_types=[
            pltpu.VMEM((_K, _SUB), jnp.int32),
            pltpu.VMEM((_K, _SUB), jnp.int32),
            pltpu.VMEM((_K, _SUB, de), f32),
            pltpu.VMEM((_K, _SUB, de), f32),
            pltpu.VMEM((_K, _SUB, de), f32),
            pltpu.VMEM((_K, _SUB, de), f32),
            pltpu.VMEM((n - (n // _NS) // 8 * 8 * (_NS - 1), de), f32),
            pltpu.VMEM_SHARED((n, de), f32),
            pltpu.VMEM_SHARED((n, de), f32),
            pltpu.SemaphoreType.DMA,
        ],
        compiler_params=pltpu.CompilerParams(use_tc_tiling_on_sc=False),
    )
    e0r3, e1r3, ag0p, ag1p = sc_fn(
        pr0, pc0, pr1, pc1, a0h, a1h, row0, col0, row1, col1)

    nb = 2000
    nblk_n = n // nb
    x_new, u_new = pl.pallas_call(
        functools.partial(_node_body, nblk_n, n),
        grid=(nblk_n,),
        in_specs=[
            pl.BlockSpec((nb, d), lambda i: (i, 0)),
            pl.BlockSpec((_NC, nb, de), lambda i: (0, i, 0)),
            pl.BlockSpec((_NC, nb, de), lambda i: (0, i, 0)),
            pl.BlockSpec((d, d), lambda i: (0, 0)),
            pl.BlockSpec((2 * de, d), lambda i: (0, 0)),
            pl.BlockSpec((1, d), lambda i: (0, 0)),
            pl.BlockSpec((d + du, du), lambda i: (0, 0)),
            pl.BlockSpec((1, du), lambda i: (0, 0)),
            pl.BlockSpec((1, du), lambda i: (0, 0)),
        ],
        out_specs=(
            pl.BlockSpec((nb, d), lambda i: (i, 0)),
            pl.BlockSpec((1, du), lambda i: (0, 0)),
        ),
        out_shape=(
            jax.ShapeDtypeStruct((n, d), f32),
            jax.ShapeDtypeStruct((1, du), f32),
        ),
        scratch_shapes=[pltpu.VMEM((1, d), f32)],
    )(x, ag0p, ag1p, wnx, wna, cn, Wg, u, bg[None, :])

    e0 = e0r3.reshape(e, de)
    e1 = e1r3.reshape(e, de)
    return (x_new, e0, e1, u_new)


# speedup vs baseline: 6.4390x; 1.1532x over previous
import functools

import jax
import jax.numpy as jnp
from jax import lax
from jax.experimental import pallas as pl
from jax.experimental.pallas import tpu as pltpu
from jax.experimental.pallas import tpu_sc as plsc

_NC = 2
_NS = 16
_NW = _NC * _NS

_CHUNK = 400
_SPLITS = ((0, 128), (128, 128), (256, 128), (384, 16))


def _prep_body(x_ref, w_ref, c_ref, p_ref):
    p = jnp.dot(x_ref[...], w_ref[...], preferred_element_type=jnp.float32)
    p_ref[...] = p + c_ref[...]


def _amat_body(ea0_ref, ea1_ref, w0_ref, w1_ref, a0_ref, a1_ref):
    a0_ref[...] = jnp.dot(ea0_ref[...], w0_ref[...],
                          preferred_element_type=jnp.float32)
    a1_ref[...] = jnp.dot(ea1_ref[...], w1_ref[...],
                          preferred_element_type=jnp.float32)


def _node_body(nblk, n_total, x_ref, a0_ref, a1_ref, wx_ref, wa_ref, cn_ref,
               wg_ref, u_ref, bg_ref, xn_ref, un_ref, acc_ref):
    i = pl.program_id(0)
    agg = jnp.concatenate(
        [a0_ref[0] + a0_ref[1], a1_ref[0] + a1_ref[1]], axis=-1)
    xb = jnp.dot(x_ref[...], wx_ref[...], preferred_element_type=jnp.float32)
    xb = xb + jnp.dot(agg, wa_ref[...], preferred_element_type=jnp.float32)
    xb = xb + cn_ref[...]
    xn_ref[...] = xb

    @pl.when(i == 0)
    def _():
        acc_ref[...] = jnp.zeros_like(acc_ref)

    acc_ref[...] += jnp.sum(xb, axis=0, keepdims=True)

    @pl.when(i == nblk - 1)
    def _():
        mean = acc_ref[...] / jnp.float32(n_total)
        un = jnp.dot(mean, wg_ref[0:128, :],
                     preferred_element_type=jnp.float32)
        un = un + jnp.dot(u_ref[...], wg_ref[128:160, :],
                          preferred_element_type=jnp.float32)
        un_ref[...] = un + bg_ref[...]


def _sc_body(n_nodes, ew, g_chunks,
             x128, a0h, a1h, row0, col0, row1, col1,
             e0p, e1p, ag0h, ag1h,
             stg, rowb, colb, rbuf, cbuf, ab, es, ebuf, cpb,
             trs, tcs, ags, sem, sem2):
    cid = lax.axis_index("c")
    sid = lax.axis_index("s")
    wid = cid * _NS + sid

    small = (n_nodes // _NS) // 8 * 8
    big = n_nodes - small * (_NS - 1)
    tile_lo = sid * small
    last = sid == _NS - 1

    z = jnp.zeros((16,), jnp.float32)

    def zero_agg():
        def zbody(i, c):
            cpb[i] = z
            return c

        lax.fori_loop(0, big, zbody, 0)

        @pl.when(last)
        def _():
            pltpu.sync_copy(cpb, ags.at[pl.ds(tile_lo, big)])

        @pl.when(jnp.logical_not(last))
        def _():
            pltpu.sync_copy(cpb.at[pl.ds(0, small)],
                            ags.at[pl.ds(tile_lo, small)])

    def copyout_agg(agh):
        @pl.when(last)
        def _():
            pltpu.sync_copy(ags.at[pl.ds(tile_lo, big)], cpb)
            pltpu.sync_copy(cpb, agh.at[cid, pl.ds(tile_lo, big)])

        @pl.when(jnp.logical_not(last))
        def _():
            cps = cpb.at[pl.ds(0, small)]
            pltpu.sync_copy(ags.at[pl.ds(tile_lo, small)], cps)
            pltpu.sync_copy(cps, agh.at[cid, pl.ds(tile_lo, small)])

    zero_agg()

    def stage_tables(colbase):
        def stage(half_rows):
            for h in range(2):
                lo = tile_lo + h * half_rows
                pltpu.sync_copy(x128.at[pl.ds(lo, half_rows)],
                                stg.at[pl.ds(0, half_rows)])
                for tab, tsp in ((0, trs), (1, tcs)):
                    def cbody(i, c, tab=tab):
                        cpb[i] = stg[i, pl.ds(colbase + tab * 16, 16)]
                        return c
                    lax.fori_loop(0, half_rows, cbody, 0)
                    pltpu.sync_copy(cpb.at[pl.ds(0, half_rows)],
                                    tsp.at[pl.ds(lo, half_rows)])

        @pl.when(last)
        def _():
            stage(big // 2)

        @pl.when(jnp.logical_not(last))
        def _():
            stage(small // 2)

    for colbase, ah, rowh, colh, ep, agh in (
        (0, a0h, row0, col0, e0p, ag0h),
        (32, a1h, row1, col1, e1p, ag1h),
    ):
        prsp, pcsp = trs, tcs
        stage_tables(colbase)
        plsc.subcore_barrier()
        ebase0 = wid * ew
        arow0 = wid * (ew // 8)

        def chunk(g, c, prsp=prsp, pcsp=pcsp, ah=ah, rowh=rowh, colh=colh,
                  ep=ep, ebase0=ebase0, arow0=arow0):
            ebase = ebase0 + g * _CHUNK
            arow = arow0 + g * (_CHUNK // 8)
            d1 = pltpu.async_copy(rowh.at[pl.ds(ebase, _CHUNK)], rowb, sem2)
            d2 = pltpu.async_copy(colh.at[pl.ds(ebase, _CHUNK)], colb, sem2)
            d3 = pltpu.async_copy(ah.at[pl.ds(arow, _CHUNK // 8)], ab, sem2)
            d1.wait()
            d2.wait()
            d3.wait()
            descs = []
            for off, sz in _SPLITS:
                descs.append(pltpu.async_copy(
                    prsp.at[rowb.at[pl.ds(off, sz)]],
                    rbuf.at[pl.ds(off, sz)], sem))
                descs.append(pltpu.async_copy(
                    pcsp.at[colb.at[pl.ds(off, sz)]],
                    cbuf.at[pl.ds(off, sz)], sem))
            for d in descs:
                d.wait()

            def ebody(j, cc):
                for t in range(8):
                    m = j * 8 + t
                    v = rbuf[m] + cbuf[m] + ab[j, pl.ds(t * 16, 16)]
                    v = jnp.maximum(v, 0.0)
                    es[m] = v
                    ebuf[j, pl.ds(t * 16, 16)] = v
                return cc

            lax.fori_loop(0, _CHUNK // 8, ebody, 0)
            pltpu.sync_copy(ebuf, ep.at[pl.ds(arow, _CHUNK // 8)])
            for off, sz in _SPLITS:
                pltpu.sync_copy(es.at[pl.ds(off, sz)],
                                ags.at[colb.at[pl.ds(off, sz)]], add=True)
            return c

        lax.fori_loop(0, g_chunks, chunk, 0)

        plsc.subcore_barrier()
        copyout_agg(agh)
        if agh is ag0h:
            zero_agg()
            plsc.subcore_barrier()


def kernel(x, edge_index_0, edge_index_1, edge_attr_0, edge_attr_1, u, batch,
           We0, be0, We1, be1, Wn, bn, Wg, bg):
    n, d = x.shape
    e = edge_index_0.shape[1]
    de = edge_attr_0.shape[1]
    du = u.shape[1]
    f32 = jnp.float32

    u0 = u[0]
    wcat = jnp.concatenate(
        [We0[:d], We0[d:2 * d], We1[:d], We1[d:2 * d],
         jnp.zeros((d, d - 4 * de), f32)], axis=1)
    c0 = u0 @ We0[2 * d + de:] + be0
    c1 = u0 @ We1[2 * d + de:] + be1
    zc = jnp.zeros_like(c0)
    cc = jnp.concatenate(
        [zc, c0, zc, c1, jnp.zeros((d - 4 * de,), f32)])[None, :]
    w8_0 = jnp.kron(jnp.eye(8, dtype=f32), We0[2 * d:2 * d + de])
    w8_1 = jnp.kron(jnp.eye(8, dtype=f32), We1[2 * d:2 * d + de])
    wnx = Wn[:d]
    wna = Wn[d:d + 2 * de]
    cn = (u0 @ Wn[d + 2 * de:] + bn)[None, :]

    x128 = pl.pallas_call(
        _prep_body,
        out_shape=jax.ShapeDtypeStruct((n, d), f32),
    )(x, wcat, cc)

    ea0r = edge_attr_0.reshape(e // 8, 8 * de)
    ea1r = edge_attr_1.reshape(e // 8, 8 * de)
    blk = 2000
    nblk_e = (e // 8) // blk
    a0r, a1r = pl.pallas_call(
        _amat_body,
        grid=(nblk_e,),
        in_specs=[
            pl.BlockSpec((blk, 8 * de), lambda i: (i, 0)),
            pl.BlockSpec((blk, 8 * de), lambda i: (i, 0)),
            pl.BlockSpec((8 * de, 8 * de), lambda i: (0, 0)),
            pl.BlockSpec((8 * de, 8 * de), lambda i: (0, 0)),
        ],
        out_specs=(
            pl.BlockSpec((blk, 8 * de), lambda i: (i, 0)),
            pl.BlockSpec((blk, 8 * de), lambda i: (i, 0)),
        ),
        out_shape=(
            jax.ShapeDtypeStruct((e // 8, 8 * de), f32),
            jax.ShapeDtypeStruct((e // 8, 8 * de), f32),
        ),
    )(ea0r, ea1r, w8_0, w8_1)

    ew = e // _NW
    g_chunks = ew // _CHUNK

    mesh = plsc.VectorSubcoreMesh(
        core_axis_name="c", subcore_axis_name="s",
        num_cores=_NC, num_subcores=_NS)
    big = n - (n // _NS) // 8 * 8 * (_NS - 1)
    sc_fn = pl.kernel(
        functools.partial(_sc_body, n, ew, g_chunks),
        out_type=(
            jax.ShapeDtypeStruct((e // 8, 8 * de), f32),
            jax.ShapeDtypeStruct((e // 8, 8 * de), f32),
            jax.ShapeDtypeStruct((_NC, n, de), f32),
            jax.ShapeDtypeStruct((_NC, n, de), f32),
        ),
        mesh=mesh,
        scratch_types=[
            pltpu.VMEM((big // 2, d), f32),
            pltpu.VMEM((_CHUNK,), jnp.int32),
            pltpu.VMEM((_CHUNK,), jnp.int32),
            pltpu.VMEM((_CHUNK, de), f32),
            pltpu.VMEM((_CHUNK, de), f32),
            pltpu.VMEM((_CHUNK // 8, d), f32),
            pltpu.VMEM((_CHUNK, de), f32),
            pltpu.VMEM((_CHUNK // 8, d), f32),
            pltpu.VMEM((big, de), f32),
            pltpu.VMEM_SHARED((n, de), f32),
            pltpu.VMEM_SHARED((n, de), f32),
            pltpu.VMEM_SHARED((n, de), f32),
            pltpu.SemaphoreType.DMA,
            pltpu.SemaphoreType.DMA,
        ],
        compiler_params=pltpu.CompilerParams(use_tc_tiling_on_sc=False),
    )
    e0p, e1p, ag0p, ag1p = sc_fn(
        x128, a0r, a1r,
        edge_index_0[0], edge_index_0[1], edge_index_1[0], edge_index_1[1])

    nb = 2000
    nblk_n = n // nb
    x_new, u_new = pl.pallas_call(
        functools.partial(_node_body, nblk_n, n),
        grid=(nblk_n,),
        in_specs=[
            pl.BlockSpec((nb, d), lambda i: (i, 0)),
            pl.BlockSpec((_NC, nb, de), lambda i: (0, i, 0)),
            pl.BlockSpec((_NC, nb, de), lambda i: (0, i, 0)),
            pl.BlockSpec((d, d), lambda i: (0, 0)),
            pl.BlockSpec((2 * de, d), lambda i: (0, 0)),
            pl.BlockSpec((1, d), lambda i: (0, 0)),
            pl.BlockSpec((d + du, du), lambda i: (0, 0)),
            pl.BlockSpec((1, du), lambda i: (0, 0)),
            pl.BlockSpec((1, du), lambda i: (0, 0)),
        ],
        out_specs=(
            pl.BlockSpec((nb, d), lambda i: (i, 0)),
            pl.BlockSpec((1, du), lambda i: (0, 0)),
        ),
        out_shape=(
            jax.ShapeDtypeStruct((n, d), f32),
            jax.ShapeDtypeStruct((1, du), f32),
        ),
        scratch_shapes=[pltpu.VMEM((1, d), f32)],
    )(x, ag0p, ag1p, wnx, wna, cn, Wg, u, bg[None, :])

    e0 = e0p.reshape(e, de)
    e1 = e1p.reshape(e, de)
    return (x_new, e0, e1, u_new)
